# Initial kernel scaffold; baseline (speedup 1.0000x reference)
import jax
import jax.numpy as jnp
from jax.experimental import pallas as pl


def kernel(final_score, depth_emb1, depth_emb2, depth1_out, depth1, depth2_out, depth2, gt_matches, epoch):
    raise NotImplementedError("write your pallas kernel here")


# speedup vs baseline: 2.3523x; 2.3523x over previous
import jax
import jax.numpy as jnp
from jax import lax
from jax.experimental import pallas as pl
from jax.experimental.pallas import tpu as pltpu

_B, _N, _D = 4, 256, 256
_H = 1024
_ROWS = _B * _H
_BLK = 512
_STEPS = _ROWS // _BLK


def _fused_body(epoch_ref, e1_ref, e2_ref, gt_ref,
                d1o_ref, d1_ref, d2o_ref, d2_ref,
                out_ref, acc_ref):
    g = pl.program_id(0)

    @pl.when(g == 0)
    def _():
        total = jnp.float32(0.0)
        count = jnp.float32(0.0)
        for b in range(_B):
            e1 = e1_ref[b]
            e2 = e2_ref[b]
            mask = (gt_ref[b] > 0).astype(jnp.float32)
            gram = lax.dot_general(e1, e2, (((1,), (1,)), ((), ())),
                                   preferred_element_type=jnp.float32)
            rc = jnp.sum(mask, axis=1)
            cc = jnp.sum(mask, axis=0)
            sqn1 = jnp.sum(e1 * e1, axis=1)
            sqn2 = jnp.sum(e2 * e2, axis=1)
            k = jnp.sum(mask)
            s = (jnp.sum(rc * sqn1) + jnp.sum(cc * sqn2)
                 - 2.0 * jnp.sum(mask * gram))
            mse = jnp.where(k > 0, s / jnp.maximum(k * jnp.float32(_D), 1.0),
                            jnp.float32(0.0))
            total = total + mse
            count = count + (k > 0).astype(jnp.float32)
        acc_ref[0] = total
        acc_ref[1] = count
        acc_ref[2] = jnp.float32(0.0)

    blk = jnp.sum((d1o_ref[...] - d1_ref[...]) ** 2) \
        + jnp.sum((d2o_ref[...] - d2_ref[...]) ** 2)
    acc_ref[2] += blk

    @pl.when(g == pl.num_programs(0) - 1)
    def _():
        depth = acc_ref[2] * jnp.float32(1.0 / (_H * _H))
        total = acc_ref[0] + jnp.where(epoch_ref[0] < 10, depth,
                                       jnp.float32(0.0))
        out_ref[0] = total / acc_ref[1]


def kernel(final_score, depth_emb1, depth_emb2, depth1_out, depth1,
           depth2_out, depth2, gt_matches, epoch):
    del final_score
    d1o = depth1_out.reshape(_ROWS, _H)
    d1 = depth1.reshape(_ROWS, _H)
    d2o = depth2_out.reshape(_ROWS, _H)
    d2 = depth2.reshape(_ROWS, _H)
    epoch_arr = jnp.asarray(epoch, jnp.int32).reshape(1)

    emb_spec = pl.BlockSpec((_B, _N, _D), lambda g: (0, 0, 0))
    gt_spec = pl.BlockSpec((_B, _N, _N), lambda g: (0, 0, 0))
    depth_spec = pl.BlockSpec((_BLK, _H), lambda g: (g, 0))

    out = pl.pallas_call(
        _fused_body,
        grid=(_STEPS,),
        in_specs=[
            pl.BlockSpec(memory_space=pltpu.SMEM),
            emb_spec, emb_spec, gt_spec,
            depth_spec, depth_spec, depth_spec, depth_spec,
        ],
        out_specs=pl.BlockSpec(memory_space=pltpu.SMEM),
        out_shape=jax.ShapeDtypeStruct((1,), jnp.float32),
        scratch_shapes=[pltpu.SMEM((3,), jnp.float32)],
    )(epoch_arr, depth_emb1, depth_emb2, gt_matches, d1o, d1, d2o, d2)
    return out.reshape(())
